# baseline (speedup 1.0000x reference)
import jax
import jax.numpy as jnp
from jax import lax
from jax.experimental import pallas as pl

_B = 16
_H = 512
_Z = 128
_A = 8
_CAND = 256
_T = 12
_TOPK = 32
_MAXA = 1.0
_MINA = -1.0
_D = 512
_BPC = 1
_NC = _B // _BPC
_R = _BPC * _CAND


def _topk_mask(returns_row):
    ii = lax.broadcasted_iota(jnp.int32, (_CAND, _CAND), 0)
    jj = lax.broadcasted_iota(jnp.int32, (_CAND, _CAND), 1)
    r_col = jnp.sum(jnp.where(ii == jj, returns_row, 0.0), axis=1, keepdims=True)
    beats = (returns_row > r_col) | ((returns_row == r_col) & (jj < ii))
    cnt = jnp.sum(beats.astype(jnp.float32), axis=1, keepdims=True)
    return (cnt < float(_TOPK)).astype(jnp.float32)


def _cem_kernel(be_ref, se_ref, n1_ref, n2_ref, Wb_ref, Ws_ref, Wa_ref,
                Wz_ref, W1_ref, w2_ref, out_ref):
    be = be_ref[0]
    se = se_ref[0]
    Wb = Wb_ref[...]
    Ws = Ws_ref[...]
    Wa = Wa_ref[...]
    Wz = Wz_ref[...]
    W1 = W1_ref[...]
    w2 = w2_ref[...]

    def rollout(get_action):
        b, s = be, se
        returns = jnp.zeros((_R, 1), jnp.float32)
        acts = []
        for t in range(_T):
            a = get_action(t)
            acts.append(a)
            pre = jnp.dot(b, Wb, preferred_element_type=jnp.float32)
            pre = pre + jnp.dot(s, Ws, preferred_element_type=jnp.float32)
            pre = pre + jnp.dot(a, Wa, preferred_element_type=jnp.float32)
            b = jnp.tanh(pre)
            s = jnp.tanh(jnp.dot(b, Wz, preferred_element_type=jnp.float32))
            h = jnp.tanh(jnp.dot(jnp.concatenate([b, s], axis=1), W1,
                                 preferred_element_type=jnp.float32))
            returns = returns + jnp.dot(h, w2,
                                        preferred_element_type=jnp.float32)
        return returns.reshape(_BPC, _CAND), acts

    inv = 1.0 / _TOPK

    returns1, acts1 = rollout(lambda t: jnp.clip(n1_ref[0, t], _MINA, _MAXA))
    m1 = jnp.concatenate(
        [_topk_mask(returns1[j:j + 1, :]) for j in range(_BPC)], axis=0)
    m13 = m1.reshape(_BPC, _CAND, 1)
    means, stds = [], []
    for t in range(_T):
        a3 = acts1[t].reshape(_BPC, _CAND, _A)
        mu = jnp.sum(a3 * m13, axis=1) * inv
        dc = a3 - mu[:, None, :]
        var = jnp.sum(dc * dc * m13, axis=1) * inv
        means.append(mu)
        stds.append(jnp.sqrt(jnp.maximum(var, 0.0)))

    def act2(t):
        mu = means[t][:, None, :]
        sd = stds[t][:, None, :]
        a3 = mu + sd * n2_ref[0, t].reshape(_BPC, _CAND, _A)
        return jnp.clip(a3, _MINA, _MAXA).reshape(_R, _A)

    returns2, acts2 = rollout(act2)
    m2 = jnp.concatenate(
        [_topk_mask(returns2[j:j + 1, :]) for j in range(_BPC)], axis=0)
    a0 = acts2[0].reshape(_BPC, _CAND, _A)
    out_ref[0] = jnp.sum(a0 * m2.reshape(_BPC, _CAND, 1), axis=1) * inv


def _plan(belief, state, Wb, Ws, Wa, Wz, W1, w2c, n1, n2, interpret=False):
    Bb, Hh = belief.shape
    Zz = state.shape[1]
    be = jnp.broadcast_to(belief[:, None, :], (Bb, _CAND, Hh)).reshape(_NC, _R, Hh)
    se = jnp.broadcast_to(state[:, None, :], (Bb, _CAND, Zz)).reshape(_NC, _R, Zz)
    out = pl.pallas_call(
        _cem_kernel,
        grid=(_NC,),
        in_specs=[
            pl.BlockSpec((1, _R, Hh), lambda i: (i, 0, 0)),
            pl.BlockSpec((1, _R, Zz), lambda i: (i, 0, 0)),
            pl.BlockSpec((1, _T, _R, _A), lambda i: (i, 0, 0, 0)),
            pl.BlockSpec((1, _T, _R, _A), lambda i: (i, 0, 0, 0)),
            pl.BlockSpec((Hh, Hh), lambda i: (0, 0)),
            pl.BlockSpec((Zz, Hh), lambda i: (0, 0)),
            pl.BlockSpec((_A, Hh), lambda i: (0, 0)),
            pl.BlockSpec((Hh, Zz), lambda i: (0, 0)),
            pl.BlockSpec((Hh + Zz, _D), lambda i: (0, 0)),
            pl.BlockSpec((_D, 1), lambda i: (0, 0)),
        ],
        out_specs=pl.BlockSpec((1, _BPC, _A), lambda i: (i, 0, 0)),
        out_shape=jax.ShapeDtypeStruct((_NC, _BPC, _A), jnp.float32),
        interpret=interpret,
    )(be, se, n1, n2, Wb, Ws, Wa, Wz, W1, w2c)
    return out.reshape(Bb, _A)


def kernel(belief, state, Wb, Ws, Wa, Wz, W1, w2):
    Bb = belief.shape[0]
    key = jax.random.key(42)
    key, s1 = jax.random.split(key)
    noise1 = jax.random.normal(s1, (_T, Bb, _CAND, _A), dtype=jnp.float32)
    key, s2 = jax.random.split(key)
    noise2 = jax.random.normal(s2, (_T, Bb, _CAND, _A), dtype=jnp.float32)
    n1 = noise1.reshape(_T, _NC, _R, _A).transpose(1, 0, 2, 3)
    n2 = noise2.reshape(_T, _NC, _R, _A).transpose(1, 0, 2, 3)
    w2c = w2.reshape(_D, 1)
    return _plan(belief, state, Wb, Ws, Wa, Wz, W1, w2c, n1, n2)


# speedup vs baseline: 1.5408x; 1.5408x over previous
import jax
import jax.numpy as jnp
from jax import lax
from jax.experimental import pallas as pl
from jax.experimental.pallas import tpu as pltpu

_B = 16
_H = 512
_Z = 128
_A = 8
_CAND = 256
_T = 12
_TOPK = 32
_MAXA = 1.0
_MINA = -1.0
_D = 512
_R = _B * _CAND
_NC = 8
_BC = _B // _NC
_RC = _R // _NC


def _topk_mask(returns):
    ii = lax.broadcasted_iota(jnp.int32, (_BC, _CAND, _CAND), 1)
    jj = lax.broadcasted_iota(jnp.int32, (_BC, _CAND, _CAND), 2)
    r_row = returns[:, None, :]
    r_col = jnp.sum(jnp.where(ii == jj, r_row, 0.0), axis=2, keepdims=True)
    beats = (r_row > r_col) | ((r_row == r_col) & (jj < ii))
    cnt = jnp.sum(beats.astype(jnp.float32), axis=2, keepdims=True)
    return (cnt < float(_TOPK)).astype(jnp.float32)


def _cem_kernel(be_ref, se_ref, n1_ref, n2_ref, Wb_ref, Ws_ref, Wa_ref,
                Wz_ref, W1_ref, w2_ref, out_ref):
    Wb = Wb_ref[...]
    Ws = Ws_ref[...]
    Wa = Wa_ref[...]
    Wz = Wz_ref[...]
    W1 = W1_ref[...]
    w2 = w2_ref[...]

    def step(b, s, a):
        pre = jnp.dot(b, Wb, preferred_element_type=jnp.float32)
        pre = pre + jnp.dot(s, Ws, preferred_element_type=jnp.float32)
        pre = pre + jnp.dot(a, Wa, preferred_element_type=jnp.float32)
        b = jnp.tanh(pre)
        s = jnp.tanh(jnp.dot(b, Wz, preferred_element_type=jnp.float32))
        h = jnp.tanh(jnp.dot(jnp.concatenate([b, s], axis=1), W1,
                             preferred_element_type=jnp.float32))
        r = jnp.dot(h, w2, preferred_element_type=jnp.float32)
        return b, s, r

    def act1(t):
        aT = n1_ref[t * _A:(t + 1) * _A, :]
        return jnp.clip(aT.T, _MINA, _MAXA)

    inv = 1.0 / _TOPK

    b = be_ref[...]
    s = se_ref[...]
    ret1 = jnp.zeros((_RC, 1), jnp.float32)
    for t in range(_T):
        b, s, r = step(b, s, act1(t))
        ret1 = ret1 + r
    m1 = _topk_mask(ret1.reshape(_BC, _CAND))

    mus = []
    sds = []
    for t in range(_T):
        a3 = act1(t).reshape(_BC, _CAND, _A)
        mu = jnp.sum(a3 * m1, axis=1) * inv
        dc = a3 - mu[:, None, :]
        var = jnp.sum(dc * dc * m1, axis=1) * inv
        mus.append(mu)
        sds.append(jnp.sqrt(jnp.maximum(var, 0.0)))

    def act2(t):
        n3 = n2_ref[t * _A:(t + 1) * _A, :].T.reshape(_BC, _CAND, _A)
        a3 = mus[t][:, None, :] + sds[t][:, None, :] * n3
        return jnp.clip(a3, _MINA, _MAXA)

    b = be_ref[...]
    s = se_ref[...]
    ret2 = jnp.zeros((_RC, 1), jnp.float32)
    a2_first = None
    for t in range(_T):
        a3 = act2(t)
        if t == 0:
            a2_first = a3
        b, s, r = step(b, s, a3.reshape(_RC, _A))
        ret2 = ret2 + r
    m2 = _topk_mask(ret2.reshape(_BC, _CAND))
    out_ref[0] = jnp.sum(a2_first * m2, axis=1) * inv


def kernel(belief, state, Wb, Ws, Wa, Wz, W1, w2):
    Bb, Hh = belief.shape
    Zz = state.shape[1]
    key = jax.random.key(42)
    key, s1 = jax.random.split(key)
    noise1 = jax.random.normal(s1, (_T, Bb, _CAND, _A), dtype=jnp.float32)
    key, s2 = jax.random.split(key)
    noise2 = jax.random.normal(s2, (_T, Bb, _CAND, _A), dtype=jnp.float32)
    n1 = noise1.reshape(_T, _R, _A).transpose(0, 2, 1).reshape(_T * _A, _R)
    n2 = noise2.reshape(_T, _R, _A).transpose(0, 2, 1).reshape(_T * _A, _R)
    be = jnp.broadcast_to(belief[:, None, :], (Bb, _CAND, Hh)).reshape(_R, Hh)
    se = jnp.broadcast_to(state[:, None, :], (Bb, _CAND, Zz)).reshape(_R, Zz)
    w2c = w2.reshape(_D, 1)
    row = lambda i: (i, 0)
    col = lambda i: (0, i)
    rep2 = lambda i: (0, 0)
    in_specs = [
        pl.BlockSpec((_RC, _H), row),
        pl.BlockSpec((_RC, _Z), row),
        pl.BlockSpec((_T * _A, _RC), col),
        pl.BlockSpec((_T * _A, _RC), col),
        pl.BlockSpec((_H, _H), rep2),
        pl.BlockSpec((_Z, _H), rep2),
        pl.BlockSpec((_A, _H), rep2),
        pl.BlockSpec((_H, _Z), rep2),
        pl.BlockSpec((_H + _Z, _D), rep2),
        pl.BlockSpec((_D, 1), rep2),
    ]
    out = pl.pallas_call(
        _cem_kernel,
        grid=(_NC,),
        in_specs=in_specs,
        out_specs=pl.BlockSpec((1, _BC, _A), lambda i: (i, 0, 0)),
        out_shape=jax.ShapeDtypeStruct((_NC, _BC, _A), jnp.float32),
    )(be, se, n1, n2, Wb, Ws, Wa, Wz, W1, w2c)
    return out.reshape(Bb, _A)
